# Initial kernel scaffold; baseline (speedup 1.0000x reference)
import jax
import jax.numpy as jnp
from jax.experimental import pallas as pl


def kernel(z, embedding):
    raise NotImplementedError("write your pallas kernel here")


# speedup vs baseline: 1.2585x; 1.2585x over previous
import functools

import jax
import jax.numpy as jnp
from jax import lax
from jax.experimental import pallas as pl
from jax.experimental.pallas import tpu as pltpu
from jax.experimental.pallas import tpu_sc as plsc

N = 8192
K = 8192
D = 32
TN = 512
TKC = 2048
NB = N // TN
NCH = K // TKC

NW = 32
TPW = N // NW
GCH = 128


def _argmin_body(z_ref, e_ref, idx_ref, loss_ref):
    pid = pl.program_id(0)
    zb = z_ref[...]
    z2 = jnp.sum(zb * zb, axis=1, keepdims=True)
    zb16 = zb.astype(jnp.bfloat16)
    minv = jnp.full((TN, 1), jnp.inf, dtype=jnp.float32)
    mini = jnp.zeros((TN, 1), dtype=jnp.int32)
    for c in range(NCH):
        ec = e_ref[c * TKC:(c + 1) * TKC, :]
        e2 = jnp.sum(ec * ec, axis=1)[None, :]
        mm = lax.dot_general(zb16, ec.astype(jnp.bfloat16),
                             (((1,), (1,)), ((), ())),
                             preferred_element_type=jnp.float32)
        d = (z2 + e2) - 2.0 * mm
        cmin = jnp.min(d, axis=1, keepdims=True)
        io = lax.broadcasted_iota(jnp.int32, (TN, TKC), 1)
        cidx = jnp.min(jnp.where(d == cmin, io, K), axis=1,
                       keepdims=True) + c * TKC
        upd = cmin < minv
        mini = jnp.where(upd, cidx, mini)
        minv = jnp.where(upd,
                         cmin.astype(jnp.bfloat16).astype(jnp.float32),
                         minv)
    idx_ref[0, 0, :] = mini[:, 0]

    @pl.when(pid == 0)
    def _():
        loss_ref[...] = jnp.zeros((1, 1), jnp.float32)

    loss_ref[...] += jnp.sum(minv).reshape(1, 1)


_argmin_call = pl.pallas_call(
    _argmin_body,
    grid=(NB,),
    in_specs=[
        pl.BlockSpec((TN, D), lambda i: (i, 0)),
        pl.BlockSpec((K, D), lambda i: (0, 0)),
    ],
    out_specs=[
        pl.BlockSpec((1, 1, TN), lambda i: (i, 0, 0)),
        pl.BlockSpec((1, 1), lambda i: (0, 0)),
    ],
    out_shape=[
        jax.ShapeDtypeStruct((NB, 1, TN), jnp.int32),
        jax.ShapeDtypeStruct((1, 1), jnp.float32),
    ],
)


def _sc_gather_hist_body(emb_hbm, idx_hbm, zvq_hbm, counts_hbm,
                         idx_v, rows_v, ones_v, zeros_v, hist_sh, sem):
    cid = lax.axis_index("c")
    sid = lax.axis_index("s")
    tbase = sid * (4 * GCH)
    for j in range(4):
        pltpu.sync_copy(idx_hbm.at[pl.ds(tbase + j * GCH, GCH)],
                        idx_v.at[j])
    for j in range(GCH // 16):
        ones_v[pl.ds(j * 16, 16)] = jnp.ones((16,), jnp.float32)
        zeros_v[pl.ds(j * 16, 16)] = jnp.zeros((16,), jnp.float32)
    for jj in range(2):
        j = cid * 2 + jj
        pltpu.async_copy(emb_hbm.at[idx_v.at[j]], rows_v, sem).wait()
        pltpu.sync_copy(rows_v, zvq_hbm.at[pl.ds(tbase + j * GCH, GCH)])
    for j in range(4):
        pltpu.sync_copy(zeros_v, hist_sh.at[pl.ds(tbase + j * GCH, GCH)])
    plsc.subcore_barrier()
    for j in range(4):
        pltpu.sync_copy(ones_v, hist_sh.at[idx_v.at[j]], add=True)
    plsc.subcore_barrier()
    obase = cid * (K // 2) + sid * (K // NW)
    pltpu.sync_copy(hist_sh.at[pl.ds(obase, K // NW)],
                    counts_hbm.at[pl.ds(obase, K // NW)])


@functools.lru_cache(maxsize=1)
def _sc_gather_hist():
    return pl.kernel(
        _sc_gather_hist_body,
        mesh=plsc.VectorSubcoreMesh(core_axis_name="c", subcore_axis_name="s"),
        out_type=[
            jax.ShapeDtypeStruct((N, D), jnp.float32),
            jax.ShapeDtypeStruct((K,), jnp.float32),
        ],
        scratch_types=[
            pltpu.VMEM((4, GCH), jnp.int32),
            pltpu.VMEM((GCH, D), jnp.float32),
            pltpu.VMEM((GCH,), jnp.float32),
            pltpu.VMEM((GCH,), jnp.float32),
            pltpu.VMEM_SHARED((K,), jnp.float32),
            pltpu.SemaphoreType.DMA,
        ],
        compiler_params=pltpu.CompilerParams(use_tc_tiling_on_sc=False),
    )


def _perplexity_body(c_ref, out_ref):
    p = c_ref[...] * (1.0 / N)
    s = jnp.sum(p * jnp.log(p + 1e-10))
    out_ref[...] = jnp.exp(-s).reshape(1, 1)


_perplexity_call = pl.pallas_call(
    _perplexity_body,
    in_specs=[pl.BlockSpec((K // 128, 128), lambda: (0, 0))],
    out_specs=pl.BlockSpec((1, 1), lambda: (0, 0)),
    out_shape=jax.ShapeDtypeStruct((1, 1), jnp.float32),
)


def kernel(z, embedding):
    B, Dz, T = z.shape
    zf = jnp.transpose(z, (0, 2, 1)).reshape(-1, Dz)
    idx3, loss11 = _argmin_call(zf, embedding)
    idx = idx3.reshape(-1)
    zvq_flat, counts = _sc_gather_hist()(embedding, idx)
    perp11 = _perplexity_call(counts.reshape(K // 128, 128))
    z_vq = jnp.transpose(zvq_flat.reshape(B, T, Dz), (0, 2, 1))
    loss = loss11[0, 0]
    perp = perp11[0, 0]
    return (z_vq, loss, loss, perp)
